# baseline (speedup 1.0000x reference)
import functools

import jax
import jax.numpy as jnp
from jax import lax
from jax.experimental import pallas as pl
from jax.experimental.pallas import tpu as pltpu
from jax.experimental.pallas import tpu_sc as plsc

DIM = 64
L = 200

NC = 2
NS = 16
NW = NC * NS

G = 128
NG = 4
C = G * NG


def _sc_embed(token_table, x_flat, aug):
  n_rows = x_flat.shape[0]
  per_w = n_rows // NW
  mesh = plsc.VectorSubcoreMesh(core_axis_name="c", subcore_axis_name="s")

  @functools.partial(
      pl.kernel,
      out_type=jax.ShapeDtypeStruct((n_rows, DIM), jnp.float32),
      mesh=mesh,
      scratch_types=[
          pltpu.VMEM((C,), jnp.int32),
          pltpu.VMEM((C,), jnp.int32),
          pltpu.VMEM((C, DIM), jnp.float32),
          pltpu.VMEM((C, DIM), jnp.float32),
          pltpu.SemaphoreType.DMA,
      ],
      compiler_params=pltpu.CompilerParams(use_tc_tiling_on_sc=False),
  )
  def k(table_hbm, x_hbm, aug_hbm, out_hbm, idx_v, pidx_v, tok_v, pos_v, sem):
    wid = lax.axis_index("s") * NC + lax.axis_index("c")
    base_w = wid * per_w

    @pl.loop(0, per_w, step=C)
    def _(off):
      base = base_w + off
      pltpu.sync_copy(x_hbm.at[pl.ds(base, C)], idx_v)

      @pl.loop(0, C, step=16)
      def _(i):
        iv = idx_v[pl.ds(i, 16)]
        p = lax.rem(base + i + lax.iota(jnp.int32, 16), L)
        pidx_v[pl.ds(i, 16)] = p + jnp.where(iv == 0, L, 0)

      copies = []
      for j in range(NG):
        copies.append(pltpu.async_copy(
            table_hbm.at[idx_v.at[pl.ds(j * G, G)]],
            tok_v.at[pl.ds(j * G, G)], sem))
        copies.append(pltpu.async_copy(
            aug_hbm.at[pidx_v.at[pl.ds(j * G, G)]],
            pos_v.at[pl.ds(j * G, G)], sem))
      for cp in copies:
        cp.wait()

      @pl.loop(0, C)
      def _(r):
        for cc in range(DIM // 16):
          slc = pl.ds(cc * 16, 16)
          plsc.addupdate(tok_v.at[r, slc], pos_v[r, slc])

      pltpu.sync_copy(tok_v, out_hbm.at[pl.ds(base, C)])

  return k(token_table, x_flat, aug)


def kernel(x, token_table, pos_table):
  Bsz, Lseq = x.shape
  x_flat = x.reshape(Bsz * Lseq)
  aug = jnp.concatenate(
      [pos_table[:Lseq], pos_table[:Lseq] - token_table[0]], axis=0)
  out = _sc_embed(token_table, x_flat, aug)
  return out.reshape(Bsz, Lseq, DIM)


# speedup vs baseline: 1.2444x; 1.2444x over previous
import functools

import jax
import jax.numpy as jnp
from jax import lax
from jax.experimental import pallas as pl
from jax.experimental.pallas import tpu as pltpu
from jax.experimental.pallas import tpu_sc as plsc

DIM = 64
L = 200

NC = 2
NS = 16
NW = NC * NS

G = 80
NG = 5
C = G * NG
NBUF = 2


def _sc_embed(token_table, x2, aug):
  n_rows = x2.shape[0] * x2.shape[1]
  per_w = n_rows // NW
  n_chunks = per_w // C
  xrows_per_chunk = C // G
  mesh = plsc.VectorSubcoreMesh(core_axis_name="c", subcore_axis_name="s")

  @functools.partial(
      pl.kernel,
      out_type=jax.ShapeDtypeStruct((n_rows, DIM), jnp.float32),
      mesh=mesh,
      scratch_types=[
          pltpu.VMEM((NG, G), jnp.int32),
          pltpu.VMEM((NG, G), jnp.int32),
          pltpu.VMEM((NG, G), jnp.int32),
          pltpu.VMEM((NG, G), jnp.int32),
          pltpu.VMEM((C, DIM), jnp.float32),
          pltpu.VMEM((C, DIM), jnp.float32),
          pltpu.VMEM((C, DIM), jnp.float32),
          pltpu.VMEM((C, DIM), jnp.float32),
          pltpu.VMEM_SHARED((2 * L, DIM), jnp.float32),
          pltpu.SemaphoreType.DMA,
          pltpu.SemaphoreType.DMA,
          pltpu.SemaphoreType.DMA,
          pltpu.SemaphoreType.DMA,
          pltpu.SemaphoreType.DMA,
          pltpu.SemaphoreType.DMA,
          pltpu.SemaphoreType.DMA,
          pltpu.SemaphoreType.DMA,
      ],
      compiler_params=pltpu.CompilerParams(use_tc_tiling_on_sc=False),
  )
  def k(table_hbm, x2_hbm, aug_hbm, out_hbm,
        idx0, idx1, pidx0, pidx1, tok0, tok1, pos0, pos1, aug_sh,
        sem_i0, sem_i1, sem_g0, sem_g1, sem_p0, sem_p1, sem_w0, sem_w1):
    sid = lax.axis_index("s")
    wid = sid * NC + lax.axis_index("c")
    base_w = wid * per_w
    xrow_w = wid * (per_w // G)

    slots = (
        (idx0, pidx0, tok0, pos0, sem_i0, sem_g0, sem_p0, sem_w0),
        (idx1, pidx1, tok1, pos1, sem_i1, sem_g1, sem_p1, sem_w1),
    )

    @pl.when(sid == 0)
    def _():
      pltpu.sync_copy(aug_hbm, aug_sh)

    plsc.subcore_barrier()

    for b in range(NBUF):
      idx_v = slots[b][0]
      sem_i = slots[b][4]
      pltpu.async_copy(
          x2_hbm.at[pl.ds(xrow_w + b * xrows_per_chunk, xrows_per_chunk)],
          idx_v, sem_i)

    iota = lax.iota(jnp.int32, 16)

    @pl.loop(0, n_chunks, step=NBUF)
    def _(g):
      for b in range(NBUF):
        idx_v, pidx_v, tok_v, pos_v, sem_i, sem_g, sem_p, sem_w = slots[b]

        @pl.when(g >= NBUF)
        def _():
          pltpu.make_async_copy(
              tok_v, out_hbm.at[pl.ds(0, C)], sem_w).wait()

        pltpu.make_async_copy(
            x2_hbm.at[pl.ds(0, xrows_per_chunk)], idx_v, sem_i).wait()

        for j in range(NG):
          for l in range(G // 16):
            pv = (j * G + l * 16) + iota
            pv = pv - jnp.where(pv >= L, L, 0)
            iv = idx_v[j, pl.ds(l * 16, 16)]
            pidx_v[j, pl.ds(l * 16, 16)] = pv + jnp.where(iv == 0, L, 0)

        for j in range(NG):
          pltpu.async_copy(
              table_hbm.at[idx_v.at[j]],
              tok_v.at[pl.ds(j * G, G)], sem_g)
          pltpu.async_copy(
              aug_sh.at[pidx_v.at[j]],
              pos_v.at[pl.ds(j * G, G)], sem_p)

      for b in range(NBUF):
        idx_v, pidx_v, tok_v, pos_v, sem_i, sem_g, sem_p, sem_w = slots[b]
        base = base_w + (g + b) * C
        for j in range(NG):
          pltpu.make_async_copy(
              table_hbm.at[idx_v.at[j]],
              tok_v.at[pl.ds(j * G, G)], sem_g).wait()
          pltpu.make_async_copy(
              aug_sh.at[pidx_v.at[j]],
              pos_v.at[pl.ds(j * G, G)], sem_p).wait()

        @pl.loop(0, C)
        def _(r):
          for k_ in range(DIM // 16):
            slc = pl.ds(k_ * 16, 16)
            plsc.addupdate(tok_v.at[r, slc], pos_v[r, slc])

        pltpu.async_copy(tok_v, out_hbm.at[pl.ds(base, C)], sem_w)

        nxt = g + b + NBUF

        @pl.when(nxt < n_chunks)
        def _():
          pltpu.async_copy(
              x2_hbm.at[pl.ds(xrow_w + nxt * xrows_per_chunk,
                              xrows_per_chunk)],
              idx_v, sem_i)

    for b in range(NBUF):
      tok_v = slots[b][2]
      sem_w = slots[b][7]
      pltpu.make_async_copy(tok_v, out_hbm.at[pl.ds(0, C)], sem_w).wait()

  return k(token_table, x2, aug)


def kernel(x, token_table, pos_table):
  Bsz, Lseq = x.shape
  x2 = x.reshape(Bsz * Lseq // G, G)
  aug = jnp.concatenate(
      [pos_table[:Lseq], pos_table[:Lseq] - token_table[0]], axis=0)
  out = _sc_embed(token_table, x2, aug)
  return out.reshape(Bsz, Lseq, DIM)
